# baseline (speedup 1.0000x reference)
import functools

import jax
import jax.numpy as jnp
from jax import lax
from jax.experimental import pallas as pl
from jax.experimental.pallas import tpu as pltpu
from jax.experimental.pallas import tpu_sc as plsc

N = 10000
E = 320000
D = 128
G = 16
NC, NS = 2, 16
NW = NC * NS
NP = 10240
CHUNK = 80
STEPS = E // NW // CHUNK
SROWS = NP // NS
FP = SROWS // CHUNK
CSTEPS = E // NS // 16
MB = 1024
GRID_M = NP // MB


def _layer_body(xl_hbm, xp_hbm, src3_hbm, dst3_hbm, out_hbm,
                idxs_v, idxd_v, rows0_v, rows1_v, sem0, sem1, agg_sh):
    cid = lax.axis_index("c")
    sid = lax.axis_index("s")
    tid = cid * NS + sid
    zero16 = jnp.zeros((16,), jnp.float32)

    for et, x_hbm in enumerate((xp_hbm, xl_hbm, xp_hbm)):
        pltpu.sync_copy(src3_hbm.at[et, tid], idxs_v)
        pltpu.sync_copy(dst3_hbm.at[et, tid], idxd_v)

        def _zrow(r, c):
            for k in range(8):
                rows0_v[r, pl.ds(k * 16, 16)] = zero16
            return c
        lax.fori_loop(0, CHUNK, _zrow, 0)
        for h in range(FP):
            pltpu.sync_copy(rows0_v,
                            agg_sh.at[pl.ds(sid * SROWS + h * CHUNK, CHUNK)])
        plsc.subcore_barrier()

        def _gather(s, buf, sem):
            pltpu.async_copy(x_hbm.at[idxs_v.at[s]], buf, sem)

        def _drain(s, buf, sem):
            pltpu.make_async_copy(x_hbm.at[idxs_v.at[s]], buf, sem).wait()
            pltpu.sync_copy(buf, agg_sh.at[idxd_v.at[s]], add=True)

        _gather(0, rows0_v, sem0)

        def _pair(i, c):
            s = 2 * i
            _gather(s + 1, rows1_v, sem1)
            _drain(s, rows0_v, sem0)
            _gather(s + 2, rows0_v, sem0)
            _drain(s + 1, rows1_v, sem1)
            return c
        lax.fori_loop(0, (STEPS - 1) // 2, _pair, 0)
        _drain(STEPS - 1, rows0_v, sem0)
        plsc.subcore_barrier()

        for h in range(FP):
            base = sid * SROWS + h * CHUNK
            pltpu.sync_copy(agg_sh.at[pl.ds(base, CHUNK)],
                            out_hbm.at[et, cid, pl.ds(base, CHUNK)])


def _sc_layer(xl_pad, xp_pad, src3_r, dst3_r):
    mesh = plsc.VectorSubcoreMesh(core_axis_name="c", subcore_axis_name="s")
    k = pl.kernel(
        _layer_body,
        out_type=jax.ShapeDtypeStruct((3, NC, NP, D), jnp.float32),
        mesh=mesh,
        scratch_types=[
            pltpu.VMEM((STEPS, CHUNK), jnp.int32),
            pltpu.VMEM((STEPS, CHUNK), jnp.int32),
            pltpu.VMEM((CHUNK, D), jnp.float32),
            pltpu.VMEM((CHUNK, D), jnp.float32),
            pltpu.SemaphoreType.DMA,
            pltpu.SemaphoreType.DMA,
            pltpu.VMEM_SHARED((NP, D), jnp.float32),
        ],
        compiler_params=pltpu.CompilerParams(use_tc_tiling_on_sc=False, needs_layout_passes=False),
    )
    return k(xl_pad, xp_pad, src3_r, dst3_r)


def _count_body(dsts_hbm, out_hbm, stg_v, hist_v, red_v, cnt_all_sh):
    cid = lax.axis_index("c")
    sid = lax.axis_index("s")
    ones = jnp.ones((16,), jnp.float32)
    zero16 = jnp.zeros((16,), jnp.float32)
    NPS = NP // NS

    for et in range(3):
        def _z(r, c):
            hist_v[pl.ds(r * 16, 16)] = zero16
            return c
        lax.fori_loop(0, NP // 16, _z, 0)

        pltpu.sync_copy(dsts_hbm.at[et, sid], stg_v)

        def _acc(j, c):
            plsc.addupdate_scatter(hist_v, [stg_v[j]], ones)
            return c
        lax.fori_loop(0, CSTEPS, _acc, 0)

        pltpu.sync_copy(hist_v, cnt_all_sh.at[sid])
        plsc.subcore_barrier()

        def _z2(r, c):
            hist_v[pl.ds(r * 16, 16)] = zero16
            return c
        lax.fori_loop(0, NPS // 16, _z2, 0)
        for t in range(NS):
            pltpu.sync_copy(cnt_all_sh.at[t, pl.ds(sid * NPS, NPS)], red_v)

            def _add(j, c):
                hist_v[pl.ds(j * 16, 16)] += red_v[pl.ds(j * 16, 16)]
                return c
            lax.fori_loop(0, NPS // 16, _add, 0)
        plsc.subcore_barrier()

        @pl.when(cid == 0)
        def _():
            pltpu.sync_copy(hist_v.at[pl.ds(0, NPS)],
                            out_hbm.at[et, pl.ds(sid * NPS, NPS)])


def _sc_counts(dsts_r):
    mesh = plsc.VectorSubcoreMesh(core_axis_name="c", subcore_axis_name="s")
    k = pl.kernel(
        _count_body,
        out_type=jax.ShapeDtypeStruct((3, NP), jnp.float32),
        mesh=mesh,
        scratch_types=[
            pltpu.VMEM((CSTEPS, 16), jnp.int32),
            pltpu.VMEM((NP,), jnp.float32),
            pltpu.VMEM((NP // NS,), jnp.float32),
            pltpu.VMEM_SHARED((NS, NP), jnp.float32),
        ],
        compiler_params=pltpu.CompilerParams(use_tc_tiling_on_sc=False, needs_layout_passes=False),
    )
    return k(dsts_r)


def _mean(a_ref, c_ref):
    a = a_ref[0, 0] + a_ref[0, 1]
    inv = 1.0 / jnp.maximum(c_ref[...], 1.0)
    return a * inv


def _dot(a, w_ref):
    return jnp.dot(a, w_ref[...], preferred_element_type=jnp.float32)


def _tc_protein_body(a1_ref, c1_ref, a2_ref, c2_ref, x_ref,
                     wl1_ref, wl2_ref, wr_ref, b_ref, o_ref):
    acc = _dot(_mean(a1_ref, c1_ref), wl1_ref)
    acc += _dot(_mean(a2_ref, c2_ref), wl2_ref)
    acc += _dot(x_ref[...], wr_ref) + b_ref[...]
    o_ref[...] = jnp.maximum(acc, 0.0)


def _tc_ligand_body(a1_ref, c1_ref, x_ref, wl1_ref, wr_ref, b_ref, o_ref):
    acc = _dot(_mean(a1_ref, c1_ref), wl1_ref)
    acc += _dot(x_ref[...], wr_ref) + b_ref[...]
    o_ref[...] = jnp.maximum(acc, 0.0)


def _pool(feat, batch_ref, o_ref):
    ids = batch_ref[...]
    onehot = (ids == lax.broadcasted_iota(jnp.int32, (1, G), 1))
    part = lax.dot_general(onehot.astype(jnp.float32), feat,
                           (((0,), (0,)), ((), ())),
                           preferred_element_type=jnp.float32)

    @pl.when(pl.program_id(0) == 0)
    def _():
        o_ref[...] = jnp.zeros_like(o_ref)
    o_ref[...] += part


def _tc_protein_pool_body(a1_ref, c1_ref, a2_ref, c2_ref, x_ref,
                          wl1_ref, wl2_ref, wr_ref, b_ref, batch_ref, o_ref):
    acc = _dot(_mean(a1_ref, c1_ref), wl1_ref)
    acc += _dot(_mean(a2_ref, c2_ref), wl2_ref)
    acc += _dot(x_ref[...], wr_ref) + b_ref[...]
    _pool(jnp.maximum(acc, 0.0), batch_ref, o_ref)


def _tc_ligand_pool_body(a1_ref, c1_ref, x_ref, wl1_ref, wr_ref, b_ref,
                         batch_ref, o_ref):
    acc = _dot(_mean(a1_ref, c1_ref), wl1_ref)
    acc += _dot(x_ref[...], wr_ref) + b_ref[...]
    _pool(jnp.maximum(acc, 0.0), batch_ref, o_ref)


def _agg3_spec(et):
    return pl.BlockSpec((1, NC, MB, D), lambda i: (et, 0, i, 0))


_CNT_SPEC = pl.BlockSpec((MB, 1), lambda i: (i, 0))
_X_SPEC = pl.BlockSpec((MB, D), lambda i: (i, 0))
_W_SPEC = pl.BlockSpec((D, D), lambda i: (0, 0))
_B_SPEC = pl.BlockSpec((1, D), lambda i: (0, 0))


def _tc_protein(agg3, c1, c2, x, wl1, wl2, wr, b):
    return pl.pallas_call(
        _tc_protein_body,
        grid=(GRID_M,),
        in_specs=[_agg3_spec(1), _CNT_SPEC, _agg3_spec(2), _CNT_SPEC, _X_SPEC,
                  _W_SPEC, _W_SPEC, _W_SPEC, _B_SPEC],
        out_specs=_X_SPEC,
        out_shape=jax.ShapeDtypeStruct((NP, D), jnp.float32),
    )(agg3, c1, agg3, c2, x, wl1, wl2, wr, b)


def _tc_ligand(agg3, c1, x, wl1, wr, b):
    return pl.pallas_call(
        _tc_ligand_body,
        grid=(GRID_M,),
        in_specs=[_agg3_spec(0), _CNT_SPEC, _X_SPEC, _W_SPEC, _W_SPEC, _B_SPEC],
        out_specs=_X_SPEC,
        out_shape=jax.ShapeDtypeStruct((NP, D), jnp.float32),
    )(agg3, c1, x, wl1, wr, b)


_BATCH_SPEC = pl.BlockSpec((MB, 1), lambda i: (i, 0))
_POOL_SPEC = pl.BlockSpec((G, D), lambda i: (0, 0))


def _tc_protein_pool(agg3, c1, c2, x, wl1, wl2, wr, b, batch):
    return pl.pallas_call(
        _tc_protein_pool_body,
        grid=(GRID_M,),
        in_specs=[_agg3_spec(1), _CNT_SPEC, _agg3_spec(2), _CNT_SPEC, _X_SPEC,
                  _W_SPEC, _W_SPEC, _W_SPEC, _B_SPEC, _BATCH_SPEC],
        out_specs=_POOL_SPEC,
        out_shape=jax.ShapeDtypeStruct((G, D), jnp.float32),
    )(agg3, c1, agg3, c2, x, wl1, wl2, wr, b, batch)


def _tc_ligand_pool(agg3, c1, x, wl1, wr, b, batch):
    return pl.pallas_call(
        _tc_ligand_pool_body,
        grid=(GRID_M,),
        in_specs=[_agg3_spec(0), _CNT_SPEC, _X_SPEC, _W_SPEC, _W_SPEC, _B_SPEC,
                  _BATCH_SPEC],
        out_specs=_POOL_SPEC,
        out_shape=jax.ShapeDtypeStruct((G, D), jnp.float32),
    )(agg3, c1, x, wl1, wr, b, batch)


def kernel(x_ligand, x_protein, ei_lp, ei_pl, ei_pp, ea_lp,
           batch_ligand, batch_protein, params):
    del ea_lp

    pad = NP - N
    xl = jnp.pad(x_ligand, ((0, pad), (0, 0)))
    xp = jnp.pad(x_protein, ((0, pad), (0, 0)))

    def _er(v):
        return v.reshape(NW, STEPS, CHUNK)

    src3 = jnp.stack([_er(ei_pl[0]), _er(ei_lp[0]), _er(ei_pp[0])])
    dst3 = jnp.stack([_er(ei_pl[1]), _er(ei_lp[1]), _er(ei_pp[1])])

    dsts_r = jnp.stack([ei_pl[1].reshape(NS, CSTEPS, 16),
                        ei_lp[1].reshape(NS, CSTEPS, 16),
                        ei_pp[1].reshape(NS, CSTEPS, 16)])
    cnts = _sc_counts(dsts_r)
    c_pl = cnts[0].reshape(NP, 1)
    c_lp = cnts[1].reshape(NP, 1)
    c_pp = cnts[2].reshape(NP, 1)

    bl = jnp.pad(batch_ligand, (0, pad), constant_values=G).reshape(NP, 1)
    bp = jnp.pad(batch_protein, (0, pad), constant_values=G).reshape(NP, 1)

    for layer in range(2):
        p = params[f"layer{layer}"]
        wl_lp, wr_lp, b_lp = p["lp"]
        wl_pp, wr_pp, b_pp = p["pp"]
        wl_pl, wr_pl, b_pl = p["pl"]
        wr_p = wr_lp + wr_pp
        b_p = (b_lp + b_pp).reshape(1, D)
        b_l = b_pl.reshape(1, D)

        agg3 = _sc_layer(xl, xp, src3, dst3)

        if layer == 0:
            new_l = _tc_ligand(agg3, c_pl, xl, wl_pl, wr_pl, b_l)
            new_p = _tc_protein(agg3, c_lp, c_pp, xp, wl_lp, wl_pp, wr_p, b_p)
            xl, xp = new_l, new_p
        else:
            pro_pool = _tc_protein_pool(agg3, c_lp, c_pp, xp,
                                        wl_lp, wl_pp, wr_p, b_p, bp)
            lig_pool = _tc_ligand_pool(agg3, c_pl, xl, wl_pl, wr_pl, b_l, bl)

    return jnp.concatenate([lig_pool, pro_pool], axis=0)


# speedup vs baseline: 1.2508x; 1.2508x over previous
import functools

import jax
import jax.numpy as jnp
from jax import lax
from jax.experimental import pallas as pl
from jax.experimental.pallas import tpu as pltpu
from jax.experimental.pallas import tpu_sc as plsc

N = 10000
E = 320000
D = 128
G = 16
NC, NS = 2, 16
NW = NC * NS
NP = 10240
CHUNK = 80
STEPS = E // NW // CHUNK
SROWS = NP // NS
FP = SROWS // CHUNK
CSTEPS = E // NS // 16
MB = 1024
GRID_M = NP // MB


def _segsum_body(x_hbm, src_hbm, dst_hbm, out_hbm,
                 idxs_v, idxd_v, rows0_v, rows1_v, sem0, sem1, agg_sh):
    cid = lax.axis_index("c")
    sid = lax.axis_index("s")
    tid = cid * NS + sid
    zero32 = jnp.zeros((32,), jnp.bfloat16)

    pltpu.sync_copy(src_hbm.at[tid], idxs_v)
    pltpu.sync_copy(dst_hbm.at[tid], idxd_v)

    def _zrow(r, c):
        for k in range(4):
            rows0_v[r, pl.ds(k * 32, 32)] = zero32
        return c
    lax.fori_loop(0, CHUNK, _zrow, 0)
    for h in range(FP):
        pltpu.sync_copy(rows0_v,
                        agg_sh.at[pl.ds(sid * SROWS + h * CHUNK, CHUNK)])
    plsc.subcore_barrier()

    def _gather(s, buf, sem):
        pltpu.async_copy(x_hbm.at[idxs_v.at[s]], buf, sem)

    def _drain(s, buf, sem):
        pltpu.make_async_copy(x_hbm.at[idxs_v.at[s]], buf, sem).wait()
        pltpu.sync_copy(buf, agg_sh.at[idxd_v.at[s]], add=True)

    _gather(0, rows0_v, sem0)

    def _pair(i, c):
        s = 2 * i
        _gather(s + 1, rows1_v, sem1)
        _drain(s, rows0_v, sem0)
        _gather(s + 2, rows0_v, sem0)
        _drain(s + 1, rows1_v, sem1)
        return c
    lax.fori_loop(0, (STEPS - 1) // 2, _pair, 0)
    _drain(STEPS - 1, rows0_v, sem0)
    plsc.subcore_barrier()

    for h in range(FP):
        base = sid * SROWS + h * CHUNK
        pltpu.sync_copy(agg_sh.at[pl.ds(base, CHUNK)],
                        out_hbm.at[cid, pl.ds(base, CHUNK)])


def _sc_segsum(x_pad, src_r, dst_r):
    mesh = plsc.VectorSubcoreMesh(core_axis_name="c", subcore_axis_name="s")
    k = pl.kernel(
        _segsum_body,
        out_type=jax.ShapeDtypeStruct((NC, NP, D), jnp.bfloat16),
        mesh=mesh,
        scratch_types=[
            pltpu.VMEM((STEPS, CHUNK), jnp.int32),
            pltpu.VMEM((STEPS, CHUNK), jnp.int32),
            pltpu.VMEM((CHUNK, D), jnp.bfloat16),
            pltpu.VMEM((CHUNK, D), jnp.bfloat16),
            pltpu.SemaphoreType.DMA,
            pltpu.SemaphoreType.DMA,
            pltpu.VMEM_SHARED((NP, D), jnp.bfloat16),
        ],
        compiler_params=pltpu.CompilerParams(use_tc_tiling_on_sc=False, needs_layout_passes=False),
    )
    return k(x_pad, src_r, dst_r)


def _count_body(dsts_hbm, out_hbm, stg_v, hist_v, red_v, cnt_all_sh):
    cid = lax.axis_index("c")
    sid = lax.axis_index("s")
    ones = jnp.ones((16,), jnp.float32)
    zero16 = jnp.zeros((16,), jnp.float32)
    NPS = NP // NS

    for et in range(3):
        def _z(r, c):
            hist_v[pl.ds(r * 16, 16)] = zero16
            return c
        lax.fori_loop(0, NP // 16, _z, 0)

        pltpu.sync_copy(dsts_hbm.at[et, sid], stg_v)

        def _acc(j, c):
            plsc.addupdate_scatter(hist_v, [stg_v[j]], ones)
            return c
        lax.fori_loop(0, CSTEPS, _acc, 0)

        pltpu.sync_copy(hist_v, cnt_all_sh.at[sid])
        plsc.subcore_barrier()

        def _z2(r, c):
            hist_v[pl.ds(r * 16, 16)] = zero16
            return c
        lax.fori_loop(0, NPS // 16, _z2, 0)
        for t in range(NS):
            pltpu.sync_copy(cnt_all_sh.at[t, pl.ds(sid * NPS, NPS)], red_v)

            def _add(j, c):
                hist_v[pl.ds(j * 16, 16)] += red_v[pl.ds(j * 16, 16)]
                return c
            lax.fori_loop(0, NPS // 16, _add, 0)
        plsc.subcore_barrier()

        @pl.when(cid == 0)
        def _():
            pltpu.sync_copy(hist_v.at[pl.ds(0, NPS)],
                            out_hbm.at[et, pl.ds(sid * NPS, NPS)])


def _sc_counts(dsts_r):
    mesh = plsc.VectorSubcoreMesh(core_axis_name="c", subcore_axis_name="s")
    k = pl.kernel(
        _count_body,
        out_type=jax.ShapeDtypeStruct((3, NP), jnp.float32),
        mesh=mesh,
        scratch_types=[
            pltpu.VMEM((CSTEPS, 16), jnp.int32),
            pltpu.VMEM((NP,), jnp.float32),
            pltpu.VMEM((NP // NS,), jnp.float32),
            pltpu.VMEM_SHARED((NS, NP), jnp.float32),
        ],
        compiler_params=pltpu.CompilerParams(use_tc_tiling_on_sc=False, needs_layout_passes=False),
    )
    return k(dsts_r)


def _mean(a_ref, c_ref):
    a = a_ref[0].astype(jnp.float32) + a_ref[1].astype(jnp.float32)
    inv = 1.0 / jnp.maximum(c_ref[...], 1.0)
    return a * inv


def _dot(a, w_ref):
    return jnp.dot(a.astype(jnp.float32), w_ref[...],
                   preferred_element_type=jnp.float32)


def _tc_protein_body(a1_ref, c1_ref, a2_ref, c2_ref, x_ref,
                     wl1_ref, wl2_ref, wr_ref, b_ref, o_ref):
    acc = _dot(_mean(a1_ref, c1_ref), wl1_ref)
    acc += _dot(_mean(a2_ref, c2_ref), wl2_ref)
    acc += _dot(x_ref[...], wr_ref) + b_ref[...]
    o_ref[...] = jnp.maximum(acc, 0.0).astype(jnp.bfloat16)


def _tc_ligand_body(a1_ref, c1_ref, x_ref, wl1_ref, wr_ref, b_ref, o_ref):
    acc = _dot(_mean(a1_ref, c1_ref), wl1_ref)
    acc += _dot(x_ref[...], wr_ref) + b_ref[...]
    o_ref[...] = jnp.maximum(acc, 0.0).astype(jnp.bfloat16)


def _pool(feat, batch_ref, o_ref):
    ids = batch_ref[...]
    onehot = (ids == lax.broadcasted_iota(jnp.int32, (1, G), 1))
    part = lax.dot_general(onehot.astype(jnp.float32), feat,
                           (((0,), (0,)), ((), ())),
                           preferred_element_type=jnp.float32)

    @pl.when(pl.program_id(0) == 0)
    def _():
        o_ref[...] = jnp.zeros_like(o_ref)
    o_ref[...] += part


def _tc_protein_pool_body(a1_ref, c1_ref, a2_ref, c2_ref, x_ref,
                          wl1_ref, wl2_ref, wr_ref, b_ref, batch_ref, o_ref):
    acc = _dot(_mean(a1_ref, c1_ref), wl1_ref)
    acc += _dot(_mean(a2_ref, c2_ref), wl2_ref)
    acc += _dot(x_ref[...], wr_ref) + b_ref[...]
    _pool(jnp.maximum(acc, 0.0), batch_ref, o_ref)


def _tc_ligand_pool_body(a1_ref, c1_ref, x_ref, wl1_ref, wr_ref, b_ref,
                         batch_ref, o_ref):
    acc = _dot(_mean(a1_ref, c1_ref), wl1_ref)
    acc += _dot(x_ref[...], wr_ref) + b_ref[...]
    _pool(jnp.maximum(acc, 0.0), batch_ref, o_ref)


_AGG_SPEC = pl.BlockSpec((NC, MB, D), lambda i: (0, i, 0))
_CNT_SPEC = pl.BlockSpec((MB, 1), lambda i: (i, 0))
_X_SPEC = pl.BlockSpec((MB, D), lambda i: (i, 0))
_W_SPEC = pl.BlockSpec((D, D), lambda i: (0, 0))
_B_SPEC = pl.BlockSpec((1, D), lambda i: (0, 0))


def _tc_protein(a1, c1, a2, c2, x, wl1, wl2, wr, b):
    return pl.pallas_call(
        _tc_protein_body,
        grid=(GRID_M,),
        in_specs=[_AGG_SPEC, _CNT_SPEC, _AGG_SPEC, _CNT_SPEC, _X_SPEC,
                  _W_SPEC, _W_SPEC, _W_SPEC, _B_SPEC],
        out_specs=_X_SPEC,
        out_shape=jax.ShapeDtypeStruct((NP, D), jnp.bfloat16),
    )(a1, c1, a2, c2, x, wl1, wl2, wr, b)


def _tc_ligand(a1, c1, x, wl1, wr, b):
    return pl.pallas_call(
        _tc_ligand_body,
        grid=(GRID_M,),
        in_specs=[_AGG_SPEC, _CNT_SPEC, _X_SPEC, _W_SPEC, _W_SPEC, _B_SPEC],
        out_specs=_X_SPEC,
        out_shape=jax.ShapeDtypeStruct((NP, D), jnp.bfloat16),
    )(a1, c1, x, wl1, wr, b)


_BATCH_SPEC = pl.BlockSpec((MB, 1), lambda i: (i, 0))
_POOL_SPEC = pl.BlockSpec((G, D), lambda i: (0, 0))


def _tc_protein_pool(a1, c1, a2, c2, x, wl1, wl2, wr, b, batch):
    return pl.pallas_call(
        _tc_protein_pool_body,
        grid=(GRID_M,),
        in_specs=[_AGG_SPEC, _CNT_SPEC, _AGG_SPEC, _CNT_SPEC, _X_SPEC,
                  _W_SPEC, _W_SPEC, _W_SPEC, _B_SPEC, _BATCH_SPEC],
        out_specs=_POOL_SPEC,
        out_shape=jax.ShapeDtypeStruct((G, D), jnp.float32),
    )(a1, c1, a2, c2, x, wl1, wl2, wr, b, batch)


def _tc_ligand_pool(a1, c1, x, wl1, wr, b, batch):
    return pl.pallas_call(
        _tc_ligand_pool_body,
        grid=(GRID_M,),
        in_specs=[_AGG_SPEC, _CNT_SPEC, _X_SPEC, _W_SPEC, _W_SPEC, _B_SPEC,
                  _BATCH_SPEC],
        out_specs=_POOL_SPEC,
        out_shape=jax.ShapeDtypeStruct((G, D), jnp.float32),
    )(a1, c1, x, wl1, wr, b, batch)


def kernel(x_ligand, x_protein, ei_lp, ei_pl, ei_pp, ea_lp,
           batch_ligand, batch_protein, params):
    del ea_lp

    pad = NP - N
    xl = jnp.pad(x_ligand, ((0, pad), (0, 0))).astype(jnp.bfloat16)
    xp = jnp.pad(x_protein, ((0, pad), (0, 0))).astype(jnp.bfloat16)

    def _er(v):
        return v.reshape(NW, STEPS, CHUNK)

    src_lp, dst_lp = _er(ei_lp[0]), _er(ei_lp[1])
    src_pp, dst_pp = _er(ei_pp[0]), _er(ei_pp[1])
    src_pl, dst_pl = _er(ei_pl[0]), _er(ei_pl[1])

    dsts_r = jnp.stack([ei_pl[1].reshape(NS, CSTEPS, 16),
                        ei_lp[1].reshape(NS, CSTEPS, 16),
                        ei_pp[1].reshape(NS, CSTEPS, 16)])
    cnts = _sc_counts(dsts_r)
    c_pl = cnts[0].reshape(NP, 1)
    c_lp = cnts[1].reshape(NP, 1)
    c_pp = cnts[2].reshape(NP, 1)

    bl = jnp.pad(batch_ligand, (0, pad), constant_values=G).reshape(NP, 1)
    bp = jnp.pad(batch_protein, (0, pad), constant_values=G).reshape(NP, 1)

    for layer in range(2):
        p = params[f"layer{layer}"]
        wl_lp, wr_lp, b_lp = p["lp"]
        wl_pp, wr_pp, b_pp = p["pp"]
        wl_pl, wr_pl, b_pl = p["pl"]
        wr_p = wr_lp + wr_pp
        b_p = (b_lp + b_pp).reshape(1, D)
        b_l = b_pl.reshape(1, D)

        agg_pl = _sc_segsum(xp, src_pl, dst_pl)
        agg_lp = _sc_segsum(xl, src_lp, dst_lp)
        agg_pp = _sc_segsum(xp, src_pp, dst_pp)

        if layer == 0:
            new_l = _tc_ligand(agg_pl, c_pl, xl, wl_pl, wr_pl, b_l)
            new_p = _tc_protein(agg_lp, c_lp, agg_pp, c_pp, xp,
                                wl_lp, wl_pp, wr_p, b_p)
            xl, xp = new_l, new_p
        else:
            pro_pool = _tc_protein_pool(agg_lp, c_lp, agg_pp, c_pp, xp,
                                        wl_lp, wl_pp, wr_p, b_p, bp)
            lig_pool = _tc_ligand_pool(agg_pl, c_pl, xl, wl_pl, wr_pl, b_l, bl)

    return jnp.concatenate([lig_pool, pro_pool], axis=0)


# speedup vs baseline: 1.4697x; 1.1750x over previous
import jax
import jax.numpy as jnp
from jax import lax
from jax.experimental import pallas as pl
from jax.experimental.pallas import tpu as pltpu
from jax.experimental.pallas import tpu_sc as plsc

N = 10000
E = 320000
D = 128
G = 16
NC, NS = 2, 16
NW = NC * NS
NPAD = 10240
CHUNK = 80
STEPS = E // NW // CHUNK
SROWS = N // NS
MB = 1000
GRID_M = N // MB

_SC_PARAMS = pltpu.CompilerParams(use_tc_tiling_on_sc=False,
                                  needs_layout_passes=False)


def _segsum_body(x_hbm, ei_hbm, out_hbm,
                 idxs_v, idxd_v, rows0_v, rows1_v, sem0, sem1, agg_sh):
    cid = lax.axis_index("c")
    sid = lax.axis_index("s")
    tid = cid * NS + sid

    pltpu.sync_copy(ei_hbm.at[0, tid], idxs_v)
    pltpu.sync_copy(ei_hbm.at[1, tid], idxd_v)

    zero32 = jnp.zeros((32,), jnp.bfloat16)

    def _zrow(r, c):
        for kk in range(4):
            rows0_v[r, pl.ds(kk * 32, 32)] = zero32
        return c
    lax.fori_loop(0, CHUNK, _zrow, 0)
    zc = SROWS // CHUNK
    for h in range(zc):
        pltpu.sync_copy(rows0_v,
                        agg_sh.at[pl.ds(sid * SROWS + h * CHUNK, CHUNK)])
    pltpu.sync_copy(
        rows0_v.at[pl.ds(0, SROWS - zc * CHUNK)],
        agg_sh.at[pl.ds(sid * SROWS + zc * CHUNK, SROWS - zc * CHUNK)])
    plsc.subcore_barrier()

    def _gather(s, buf, sem):
        pltpu.async_copy(x_hbm.at[idxs_v.at[s]], buf, sem)

    def _drain(s, buf, sem):
        pltpu.make_async_copy(x_hbm.at[idxs_v.at[s]], buf, sem).wait()
        pltpu.sync_copy(buf, agg_sh.at[idxd_v.at[s]], add=True)

    _gather(0, rows0_v, sem0)

    def _pair(i, c):
        s = 2 * i
        _gather(s + 1, rows1_v, sem1)
        _drain(s, rows0_v, sem0)
        _gather(s + 2, rows0_v, sem0)
        _drain(s + 1, rows1_v, sem1)
        return c
    lax.fori_loop(0, (STEPS - 1) // 2, _pair, 0)
    _drain(STEPS - 1, rows0_v, sem0)
    plsc.subcore_barrier()

    pltpu.sync_copy(agg_sh.at[pl.ds(sid * SROWS, SROWS)],
                    out_hbm.at[cid, pl.ds(sid * SROWS, SROWS)])


def _sc_segsum(x_bf, ei_r):
    mesh = plsc.VectorSubcoreMesh(core_axis_name="c", subcore_axis_name="s")
    k = pl.kernel(
        _segsum_body,
        out_type=jax.ShapeDtypeStruct((NC, N, D), jnp.bfloat16),
        mesh=mesh,
        scratch_types=[
            pltpu.VMEM((STEPS, CHUNK), jnp.int32),
            pltpu.VMEM((STEPS, CHUNK), jnp.int32),
            pltpu.VMEM((CHUNK, D), jnp.bfloat16),
            pltpu.VMEM((CHUNK, D), jnp.bfloat16),
            pltpu.SemaphoreType.DMA,
            pltpu.SemaphoreType.DMA,
            pltpu.VMEM_SHARED((N, D), jnp.bfloat16),
        ],
        compiler_params=_SC_PARAMS,
    )
    return k(x_bf, ei_r)


def _count_body(ei0_hbm, ei1_hbm, ei2_hbm, out_hbm,
                stg_v, hist_v, red_v, cnt_all_sh):
    cid = lax.axis_index("c")
    sid = lax.axis_index("s")
    tid = cid * NS + sid
    ones = jnp.ones((16,), jnp.float32)
    zero16 = jnp.zeros((16,), jnp.float32)
    NPS = NPAD // NS

    for et, ei_hbm in enumerate((ei0_hbm, ei1_hbm, ei2_hbm)):
        def _z(r, c):
            hist_v[pl.ds(r * 16, 16)] = zero16
            return c
        lax.fori_loop(0, NPAD // 16, _z, 0)

        pltpu.sync_copy(ei_hbm.at[1, tid], stg_v)

        def _acc(j, c):
            for kk in range(CHUNK // 16):
                plsc.addupdate_scatter(
                    hist_v, [stg_v[j, pl.ds(kk * 16, 16)]], ones)
            return c
        lax.fori_loop(0, STEPS, _acc, 0)

        pltpu.sync_copy(hist_v, cnt_all_sh.at[sid])
        plsc.subcore_barrier()

        def _z2(r, c):
            hist_v[pl.ds(r * 16, 16)] = zero16
            return c
        lax.fori_loop(0, NPS // 16, _z2, 0)
        for t in range(NS):
            pltpu.sync_copy(cnt_all_sh.at[t, pl.ds(sid * NPS, NPS)], red_v)

            def _add(j, c):
                hist_v[pl.ds(j * 16, 16)] += red_v[pl.ds(j * 16, 16)]
                return c
            lax.fori_loop(0, NPS // 16, _add, 0)
        plsc.subcore_barrier()

        pltpu.sync_copy(hist_v.at[pl.ds(0, NPS)],
                        out_hbm.at[et, cid, pl.ds(sid * NPS, NPS)])


def _sc_counts(ei0_r, ei1_r, ei2_r):
    mesh = plsc.VectorSubcoreMesh(core_axis_name="c", subcore_axis_name="s")
    k = pl.kernel(
        _count_body,
        out_type=jax.ShapeDtypeStruct((3, NC, NPAD), jnp.float32),
        mesh=mesh,
        scratch_types=[
            pltpu.VMEM((STEPS, CHUNK), jnp.int32),
            pltpu.VMEM((NPAD,), jnp.float32),
            pltpu.VMEM((NPAD // NS,), jnp.float32),
            pltpu.VMEM_SHARED((NS, NPAD), jnp.float32),
        ],
        compiler_params=_SC_PARAMS,
    )
    return k(ei0_r, ei1_r, ei2_r)


def _mean(a_ref, c_ref):
    a = a_ref[0].astype(jnp.float32) + a_ref[1].astype(jnp.float32)
    cnt = c_ref[0, 0] + c_ref[0, 1]
    inv = 1.0 / jnp.maximum(cnt, 1.0)
    return a * inv


def _dot(a, w_ref):
    return jnp.dot(a.astype(jnp.float32), w_ref[...],
                   preferred_element_type=jnp.float32)


def _protein_acc(a1_ref, c1_ref, a2_ref, c2_ref, x_ref,
                 wl1_ref, wl2_ref, wr_ref, b_ref):
    acc = _dot(_mean(a1_ref, c1_ref), wl1_ref)
    acc += _dot(_mean(a2_ref, c2_ref), wl2_ref)
    acc += _dot(x_ref[...], wr_ref) + b_ref[...]
    return jnp.maximum(acc, 0.0)


def _ligand_acc(a1_ref, c1_ref, x_ref, wl1_ref, wr_ref, b_ref):
    acc = _dot(_mean(a1_ref, c1_ref), wl1_ref)
    acc += _dot(x_ref[...], wr_ref) + b_ref[...]
    return jnp.maximum(acc, 0.0)


def _tc_protein_body(a1_ref, c1_ref, a2_ref, c2_ref, x_ref,
                     wl1_ref, wl2_ref, wr_ref, b_ref, o_ref):
    o_ref[...] = _protein_acc(a1_ref, c1_ref, a2_ref, c2_ref, x_ref,
                              wl1_ref, wl2_ref, wr_ref,
                              b_ref).astype(jnp.bfloat16)


def _tc_ligand_body(a1_ref, c1_ref, x_ref, wl1_ref, wr_ref, b_ref, o_ref):
    o_ref[...] = _ligand_acc(a1_ref, c1_ref, x_ref, wl1_ref, wr_ref,
                             b_ref).astype(jnp.bfloat16)


def _pool(feat, batch_ref, o_ref):
    ids = batch_ref[...]
    onehot = (ids == lax.broadcasted_iota(jnp.int32, (1, G), 1))
    part = lax.dot_general(onehot.astype(jnp.float32), feat,
                           (((0,), (0,)), ((), ())),
                           preferred_element_type=jnp.float32)

    @pl.when(pl.program_id(0) == 0)
    def _():
        o_ref[...] = jnp.zeros_like(o_ref)
    o_ref[...] += part


def _tc_protein_pool_body(a1_ref, c1_ref, a2_ref, c2_ref, x_ref,
                          wl1_ref, wl2_ref, wr_ref, b_ref, batch_ref, o_ref):
    _pool(_protein_acc(a1_ref, c1_ref, a2_ref, c2_ref, x_ref,
                       wl1_ref, wl2_ref, wr_ref, b_ref), batch_ref, o_ref)


def _tc_ligand_pool_body(a1_ref, c1_ref, x_ref, wl1_ref, wr_ref, b_ref,
                         batch_ref, o_ref):
    _pool(_ligand_acc(a1_ref, c1_ref, x_ref, wl1_ref, wr_ref, b_ref),
          batch_ref, o_ref)


_AGG_SPEC = pl.BlockSpec((NC, MB, D), lambda i: (0, i, 0))
_X_SPEC = pl.BlockSpec((MB, D), lambda i: (i, 0))
_W_SPEC = pl.BlockSpec((D, D), lambda i: (0, 0))
_B_SPEC = pl.BlockSpec((1, D), lambda i: (0, 0))
_BATCH_SPEC = pl.BlockSpec((MB, 1), lambda i: (i, 0))
_POOL_SPEC = pl.BlockSpec((G, D), lambda i: (0, 0))


def _cnt_spec(et):
    return pl.BlockSpec((1, NC, MB, 1), lambda i: (et, 0, i, 0))


def _tc_protein(a1, a2, cnts, x, wl1, wl2, wr, b):
    return pl.pallas_call(
        _tc_protein_body,
        grid=(GRID_M,),
        in_specs=[_AGG_SPEC, _cnt_spec(1), _AGG_SPEC, _cnt_spec(2), _X_SPEC,
                  _W_SPEC, _W_SPEC, _W_SPEC, _B_SPEC],
        out_specs=_X_SPEC,
        out_shape=jax.ShapeDtypeStruct((N, D), jnp.bfloat16),
    )(a1, cnts, a2, cnts, x, wl1, wl2, wr, b)


def _tc_ligand(a1, cnts, x, wl1, wr, b):
    return pl.pallas_call(
        _tc_ligand_body,
        grid=(GRID_M,),
        in_specs=[_AGG_SPEC, _cnt_spec(0), _X_SPEC, _W_SPEC, _W_SPEC,
                  _B_SPEC],
        out_specs=_X_SPEC,
        out_shape=jax.ShapeDtypeStruct((N, D), jnp.bfloat16),
    )(a1, cnts, x, wl1, wr, b)


def _tc_protein_pool(a1, a2, cnts, x, wl1, wl2, wr, b, batch):
    return pl.pallas_call(
        _tc_protein_pool_body,
        grid=(GRID_M,),
        in_specs=[_AGG_SPEC, _cnt_spec(1), _AGG_SPEC, _cnt_spec(2), _X_SPEC,
                  _W_SPEC, _W_SPEC, _W_SPEC, _B_SPEC, _BATCH_SPEC],
        out_specs=_POOL_SPEC,
        out_shape=jax.ShapeDtypeStruct((G, D), jnp.float32),
    )(a1, cnts, a2, cnts, x, wl1, wl2, wr, b, batch)


def _tc_ligand_pool(a1, cnts, x, wl1, wr, b, batch):
    return pl.pallas_call(
        _tc_ligand_pool_body,
        grid=(GRID_M,),
        in_specs=[_AGG_SPEC, _cnt_spec(0), _X_SPEC, _W_SPEC, _W_SPEC,
                  _B_SPEC, _BATCH_SPEC],
        out_specs=_POOL_SPEC,
        out_shape=jax.ShapeDtypeStruct((G, D), jnp.float32),
    )(a1, cnts, x, wl1, wr, b, batch)


def kernel(x_ligand, x_protein, ei_lp, ei_pl, ei_pp, ea_lp,
           batch_ligand, batch_protein, params):
    del ea_lp

    xl = x_ligand.astype(jnp.bfloat16)
    xp = x_protein.astype(jnp.bfloat16)

    def _er(ei):
        return ei.reshape(2, NW, STEPS, CHUNK)

    ei_pl_r, ei_lp_r, ei_pp_r = _er(ei_pl), _er(ei_lp), _er(ei_pp)

    cnts = _sc_counts(ei_pl_r, ei_lp_r, ei_pp_r)
    cnts4 = cnts.reshape(3, NC, NPAD, 1)

    bl = batch_ligand.reshape(N, 1)
    bp = batch_protein.reshape(N, 1)

    for layer in range(2):
        p = params[f"layer{layer}"]
        wl_lp, wr_lp, b_lp = p["lp"]
        wl_pp, wr_pp, b_pp = p["pp"]
        wl_pl, wr_pl, b_pl = p["pl"]
        wr_p = wr_lp + wr_pp
        b_p = (b_lp + b_pp).reshape(1, D)
        b_l = b_pl.reshape(1, D)

        agg_pl = _sc_segsum(xp, ei_pl_r)
        agg_lp = _sc_segsum(xl, ei_lp_r)
        agg_pp = _sc_segsum(xp, ei_pp_r)

        if layer == 0:
            new_l = _tc_ligand(agg_pl, cnts4, xl, wl_pl, wr_pl, b_l)
            new_p = _tc_protein(agg_lp, agg_pp, cnts4, xp,
                                wl_lp, wl_pp, wr_p, b_p)
            xl, xp = new_l, new_p
        else:
            pro_pool = _tc_protein_pool(agg_lp, agg_pp, cnts4, xp,
                                        wl_lp, wl_pp, wr_p, b_p, bp)
            lig_pool = _tc_ligand_pool(agg_pl, cnts4, xl, wl_pl, wr_pl,
                                       b_l, bl)

    return jnp.concatenate([lig_pool, pro_pool], axis=0)
